# baseline (speedup 1.0000x reference)
import functools
import jax
import jax.numpy as jnp
import numpy as np
from jax import lax
from jax.experimental import pallas as pl
from jax.experimental.pallas import tpu as pltpu
from jax.experimental.pallas import tpu_sc as plsc

_B, _N, _CIN, _COUT = 8, 1000000, 4, 3
_NPOINTS = _B * _N
_NW = 32
_PTS_PER_W = _NPOINTS // _NW
_PC = 2000
_CHUNKS = _PTS_PER_W // _PC
_STEPS = _PC // 16

_mesh = plsc.VectorSubcoreMesh(core_axis_name="c", subcore_axis_name="s")


@functools.partial(
    pl.kernel,
    mesh=_mesh,
    out_type=jax.ShapeDtypeStruct((_NPOINTS * _COUT,), jnp.float32),
    scratch_types=[
        pltpu.VMEM((_CIN * _PC,), jnp.float32),
        pltpu.VMEM((_COUT * _PC,), jnp.float32),
        pltpu.VMEM((48,), jnp.int32),
    ],
    compiler_params=pltpu.CompilerParams(needs_layout_passes=False),
)
def _sph2vec_sc(x_hbm, idx_hbm, out_hbm, in_v, out_v, idx_v):
    c = lax.axis_index("c")
    s = lax.axis_index("s")
    wid = s * 2 + c
    in_base = wid * (_PTS_PER_W * _CIN)
    out_base = wid * (_PTS_PER_W * _COUT)

    pltpu.sync_copy(idx_hbm, idx_v)
    idx_t = [idx_v[pl.ds(16 * t, 16)] for t in range(3)]

    def chunk_body(ci, carry):
        pltpu.sync_copy(
            x_hbm.at[pl.ds(in_base + ci * (_CIN * _PC), _CIN * _PC)], in_v
        )

        def step(si, carry2):
            base_in = si * 64
            base_out = si * 48
            for t in range(3):
                v = plsc.load_gather(in_v, [idx_t[t] + base_in])
                out_v[pl.ds(base_out + 16 * t, 16)] = v
            return carry2

        lax.fori_loop(0, _STEPS, step, 0, unroll=2)
        pltpu.sync_copy(
            out_v, out_hbm.at[pl.ds(out_base + ci * (_COUT * _PC), _COUT * _PC)]
        )
        return carry

    lax.fori_loop(0, _CHUNKS, chunk_body, 0)


_P = np.array([3, 1, 2], dtype=np.int32)
_O = np.arange(48, dtype=np.int32)
_IDX48 = _CIN * (_O // 3) + _P[_O % 3]


def kernel(x):
    flat = x.reshape(-1)
    out = _sph2vec_sc(flat, _IDX48)
    return out.reshape(_B, _N, _COUT)


# speedup vs baseline: 136.3788x; 136.3788x over previous
import functools
import jax
import jax.numpy as jnp
from jax import lax
from jax.experimental import pallas as pl
from jax.experimental.pallas import tpu as pltpu
from jax.experimental.pallas import tpu_sc as plsc

_B, _N, _CIN, _COUT = 8, 1000000, 4, 3
_NW = 32
_CL = 1024
_NU = _N // _CL
_TAIL0 = _NU * _CL
_TAILLEN = _N - _TAIL0
_TAILW = _NU % _NW
_PSRC = (3, 1, 2)

_mesh = plsc.VectorSubcoreMesh(core_axis_name="c", subcore_axis_name="s")


@functools.partial(
    pl.kernel,
    mesh=_mesh,
    out_type=jax.ShapeDtypeStruct((_COUT, _B, _N), jnp.float32),
    scratch_types=[
        pltpu.VMEM((_B, _CIN, _CL), jnp.float32),
        pltpu.VMEM((_B, _CIN, _TAILLEN), jnp.float32),
    ],
    compiler_params=pltpu.CompilerParams(needs_layout_passes=False),
)
def _sph2vec_sc(xt_hbm, out_hbm, in_v, tail_v):
    cid = lax.axis_index("c")
    sid = lax.axis_index("s")
    wid = sid * 2 + cid
    n_units = (_NU - wid + _NW - 1) // _NW

    def unit_body(u, carry):
        n0 = (u * _NW + wid) * _CL
        pltpu.sync_copy(xt_hbm.at[:, :, pl.ds(n0, _CL)], in_v)
        for ch in range(_COUT):
            pltpu.sync_copy(
                in_v.at[:, _PSRC[ch], :], out_hbm.at[ch, :, pl.ds(n0, _CL)]
            )
        return carry

    lax.fori_loop(0, n_units, unit_body, 0)

    @pl.when(wid == _TAILW)
    def _tail():
        pltpu.sync_copy(xt_hbm.at[:, :, pl.ds(_TAIL0, _TAILLEN)], tail_v)
        for ch in range(_COUT):
            pltpu.sync_copy(
                tail_v.at[:, _PSRC[ch], :],
                out_hbm.at[ch, :, pl.ds(_TAIL0, _TAILLEN)],
            )


def kernel(x):
    xt = jnp.transpose(x, (0, 2, 1))
    out_t = _sph2vec_sc(xt)
    return jnp.transpose(out_t, (1, 2, 0))
